# baseline (speedup 1.0000x reference)
import functools

import jax
import jax.numpy as jnp
from jax import lax
from jax.experimental import pallas as pl
from jax.experimental.pallas import tpu as pltpu
from jax.experimental.pallas import tpu_sc as plsc

_B = 64
_N_IN = 512
_N_OUT = 1024

_NC = 2
_NS = 16
_NW = _NC * _NS
_BPW = _B // _NW

_ZR = 32
_ZTOP = 16


def _fill_zeros(ref, nrows, ncols):
    z = jnp.zeros((16,), jnp.float32)

    def row(r, carry):
        for c in range(0, ncols, 16):
            ref[r, pl.ds(c, 16)] = z
        return carry

    lax.fori_loop(0, nrows, row, 0)


def _sc_insert(rho_hbm, out_hbm, zfull, asm0, asm1, asm2,
               zsem, gsem0, gsem1, gsem2, ssem0, ssem1, ssem2):
    wid = lax.axis_index("s") * _NC + lax.axis_index("c")

    asm = (asm0, asm1, asm2)
    gsem = (gsem0, gsem1, gsem2)
    ssem = (ssem0, ssem1, ssem2)
    nbuf = len(asm)

    batches = [wid * _BPW + bi for bi in range(_BPW)]
    chunks = [(b, r) for b in batches for r in range(0, _N_IN, _ZR)]
    n = len(chunks)

    gd = [None] * nbuf
    sd = [None] * nbuf

    def issue_gather(i):
        b, r = chunks[i]
        buf = i % nbuf
        if sd[buf] is not None:
            sd[buf].wait()
            sd[buf] = None
        gd[buf] = pltpu.async_copy(
            rho_hbm.at[b, pl.ds(r, _ZR)],
            asm[buf].at[:, pl.ds(_N_IN, _N_IN)],
            gsem[buf],
        )

    _fill_zeros(zfull, _ZTOP, _N_OUT)
    for a in asm:
        _fill_zeros(a, _ZR, _N_IN)

    for i in range(min(nbuf, n)):
        issue_gather(i)

    zcopies = []
    for b in batches:
        for r in range(0, _N_IN, _ZTOP):
            zcopies.append(
                pltpu.async_copy(zfull, out_hbm.at[b, pl.ds(r, _ZTOP)], zsem)
            )

    for i in range(n):
        buf = i % nbuf
        gd[buf].wait()
        b, r = chunks[i]
        sd[buf] = pltpu.async_copy(
            asm[buf], out_hbm.at[b, pl.ds(_N_IN + r, _ZR)], ssem[buf]
        )
        if i + nbuf < n:
            issue_gather(i + nbuf)

    for buf in range(nbuf):
        if sd[buf] is not None:
            sd[buf].wait()
    for c in zcopies:
        c.wait()


def kernel(rho):
    sc_call = functools.partial(
        pl.kernel,
        out_type=jax.ShapeDtypeStruct((_B, _N_OUT, _N_OUT), jnp.float32),
        mesh=plsc.VectorSubcoreMesh(
            core_axis_name="c", subcore_axis_name="s",
            num_cores=_NC, num_subcores=_NS,
        ),
        scratch_types=[
            pltpu.VMEM((_ZTOP, _N_OUT), jnp.float32),
            pltpu.VMEM((_ZR, _N_OUT), jnp.float32),
            pltpu.VMEM((_ZR, _N_OUT), jnp.float32),
            pltpu.VMEM((_ZR, _N_OUT), jnp.float32),
            pltpu.SemaphoreType.DMA,
            pltpu.SemaphoreType.DMA,
            pltpu.SemaphoreType.DMA,
            pltpu.SemaphoreType.DMA,
            pltpu.SemaphoreType.DMA,
            pltpu.SemaphoreType.DMA,
            pltpu.SemaphoreType.DMA,
        ],
    )(_sc_insert)
    return sc_call(rho)


# speedup vs baseline: 1.0600x; 1.0600x over previous
import functools

import jax
import jax.numpy as jnp
from jax import lax
from jax.experimental import pallas as pl
from jax.experimental.pallas import tpu as pltpu
from jax.experimental.pallas import tpu_sc as plsc

_B = 64
_N_IN = 512
_N_OUT = 1024

_NC = 2
_NS = 16
_NW = _NC * _NS
_BPW = _B // _NW

_ZTOP = 16
_ZLEFT = 64
_QR = 64


def _fill_zeros(ref, nrows, ncols):
    z = jnp.zeros((16,), jnp.float32)

    def row(r, carry):
        for c in range(0, ncols, 16):
            ref[r, pl.ds(c, 16)] = z
        return carry

    lax.fori_loop(0, nrows, row, 0)


def _sc_insert(rho_hbm, out_hbm, zfull, zhalf, stg0, stg1,
               zsem, gsem0, gsem1, ssem0, ssem1):
    wid = lax.axis_index("s") * _NC + lax.axis_index("c")

    stg = (stg0, stg1)
    gsem = (gsem0, gsem1)
    ssem = (ssem0, ssem1)

    batches = [wid * _BPW + bi for bi in range(_BPW)]
    chunks = [(b, r) for b in batches for r in range(0, _N_IN, _QR)]
    n = len(chunks)

    gd = [None, None]
    sd = [None, None]

    def issue_gather(i):
        b, r = chunks[i]
        buf = i % 2
        if sd[buf] is not None:
            sd[buf].wait()
            sd[buf] = None
        gd[buf] = pltpu.async_copy(
            rho_hbm.at[b, pl.ds(r, _QR)], stg[buf], gsem[buf]
        )

    issue_gather(0)
    issue_gather(1)

    _fill_zeros(zfull, _ZTOP, _N_OUT)
    _fill_zeros(zhalf, _ZLEFT, _N_IN)

    zcopies = []
    for b in batches:
        for r in range(0, _N_IN, _ZTOP):
            zcopies.append(
                pltpu.async_copy(zfull, out_hbm.at[b, pl.ds(r, _ZTOP)], zsem)
            )
        for r in range(0, _N_IN, _ZLEFT):
            zcopies.append(
                pltpu.async_copy(
                    zhalf,
                    out_hbm.at[b, pl.ds(_N_IN + r, _ZLEFT), pl.ds(0, _N_IN)],
                    zsem,
                )
            )

    for i in range(n):
        buf = i % 2
        gd[buf].wait()
        b, r = chunks[i]
        sd[buf] = pltpu.async_copy(
            stg[buf],
            out_hbm.at[b, pl.ds(_N_IN + r, _QR), pl.ds(_N_IN, _N_IN)],
            ssem[buf],
        )
        if i + 2 < n:
            issue_gather(i + 2)

    for buf in (0, 1):
        if sd[buf] is not None:
            sd[buf].wait()
    for c in zcopies:
        c.wait()


def kernel(rho):
    sc_call = functools.partial(
        pl.kernel,
        out_type=jax.ShapeDtypeStruct((_B, _N_OUT, _N_OUT), jnp.float32),
        mesh=plsc.VectorSubcoreMesh(
            core_axis_name="c", subcore_axis_name="s",
            num_cores=_NC, num_subcores=_NS,
        ),
        scratch_types=[
            pltpu.VMEM((_ZTOP, _N_OUT), jnp.float32),
            pltpu.VMEM((_ZLEFT, _N_IN), jnp.float32),
            pltpu.VMEM((_QR, _N_IN), jnp.float32),
            pltpu.VMEM((_QR, _N_IN), jnp.float32),
            pltpu.SemaphoreType.DMA,
            pltpu.SemaphoreType.DMA,
            pltpu.SemaphoreType.DMA,
            pltpu.SemaphoreType.DMA,
            pltpu.SemaphoreType.DMA,
        ],
    )(_sc_insert)
    return sc_call(rho)
